# baseline (speedup 1.0000x reference)
import functools

import jax
import jax.numpy as jnp
from jax import lax
from jax.experimental import pallas as pl
from jax.experimental.pallas import tpu as pltpu
from jax.experimental.pallas import tpu_sc as plsc

B = 16384
VOCAB = 150
EMB = 32

_NC, _NS = 2, 16
_NW = _NC * _NS
_C = B // _NW
_G = _C // 16


def _sc_pool_body(idx_hbm, table_hbm, out_hbm, idx_v, table_v, out_v):
    wid = lax.axis_index("s") * _NC + lax.axis_index("c")
    base = wid * _C
    pltpu.sync_copy(table_hbm, table_v)
    pltpu.sync_copy(idx_hbm.at[:, pl.ds(base, _C)], idx_v)

    def group(g, carry):
        off = g * 16
        ivs = [idx_v[s, pl.ds(off, 16)] for s in range(10)]
        for f in range(EMB):
            col = jnp.full((16,), f, jnp.int32)
            acc_r = plsc.load_gather(table_v, [ivs[0], col])
            for s in range(1, 5):
                acc_r = acc_r + plsc.load_gather(table_v, [ivs[s], col])
            acc_d = plsc.load_gather(table_v, [ivs[5], col])
            for s in range(6, 10):
                acc_d = acc_d + plsc.load_gather(table_v, [ivs[s], col])
            out_v[f, pl.ds(off, 16)] = acc_r
            out_v[EMB + f, pl.ds(off, 16)] = acc_d
        return carry

    lax.fori_loop(0, _G, group, 0)
    pltpu.sync_copy(out_v, out_hbm.at[:, pl.ds(base, _C)])


@functools.cache
def _sc_pool():
    return pl.kernel(
        _sc_pool_body,
        out_type=jax.ShapeDtypeStruct((2 * EMB, B), jnp.float32),
        mesh=plsc.VectorSubcoreMesh(
            core_axis_name="c", subcore_axis_name="s", num_cores=_NC, num_subcores=_NS
        ),
        scratch_types=[
            pltpu.VMEM((10, _C), jnp.int32),
            pltpu.VMEM((VOCAB, EMB + 1), jnp.float32),
            pltpu.VMEM((2 * EMB, _C), jnp.float32),
        ],
        compiler_params=pltpu.CompilerParams(needs_layout_passes=False),
    )


_BLK = 2048


def _mlp_body(x_ref, w1t_ref, b1_ref, w2t_ref, b2_ref, w3t_ref, b3_ref, o_ref):
    x = x_ref[...]
    h = jnp.dot(w1t_ref[...], x, preferred_element_type=jnp.float32) + b1_ref[...]
    h = jnp.maximum(h, 0.0)
    h = jnp.dot(w2t_ref[...], h, preferred_element_type=jnp.float32) + b2_ref[...]
    h = jnp.maximum(h, 0.0)
    y = jnp.dot(w3t_ref[...], h, preferred_element_type=jnp.float32) + b3_ref[...]
    o_ref[...] = 1.0 / (1.0 + jnp.exp(-y))


def _mlp(xT, W1T, b1c, W2T, b2c, W3T, b3c):
    grid = B // _BLK
    full = lambda shape: pl.BlockSpec(shape, lambda i: (0, 0))
    return pl.pallas_call(
        _mlp_body,
        grid=(grid,),
        in_specs=[
            pl.BlockSpec((2 * EMB, _BLK), lambda i: (0, i)),
            full(W1T.shape), full(b1c.shape),
            full(W2T.shape), full(b2c.shape),
            full(W3T.shape), full(b3c.shape),
        ],
        out_specs=pl.BlockSpec((1, _BLK), lambda i: (0, i)),
        out_shape=jax.ShapeDtypeStruct((1, B), jnp.float32),
    )(xT, W1T, b1c, W2T, b2c, W3T, b3c)


def kernel(dire_heros, radiant_heros, embed_table, W1, b1, W2, b2, W3, b3):
    idxT = jnp.concatenate(
        [radiant_heros.astype(jnp.int32).T, dire_heros.astype(jnp.int32).T], axis=0
    )
    table_p = jnp.pad(embed_table, ((0, 0), (0, 1)))
    xT = _sc_pool()(idxT, table_p)
    y = _mlp(xT, W1.T, b1[:, None], W2.T, b2[:, None], W3.T, b3[:, None])
    return y.T


# speedup vs baseline: 2.5952x; 2.5952x over previous
import functools

import jax
import jax.numpy as jnp
from jax import lax
from jax.experimental import pallas as pl
from jax.experimental.pallas import tpu as pltpu
from jax.experimental.pallas import tpu_sc as plsc

B = 16384
VOCAB = 150
EMB = 32

_NC, _NS = 2, 16
_NW = _NC * _NS
_C = B // _NW
_G = _C // 16


_COPY = VOCAB * EMB + 1
_TWORDS = 16 * _COPY


def _sc_pool_body(idx_hbm, table_hbm, out_hbm, idx_v, table_v, out_v):
    wid = lax.axis_index("s") * _NC + lax.axis_index("c")
    base = wid * _C
    pltpu.sync_copy(table_hbm, table_v)
    pltpu.sync_copy(idx_hbm.at[:, pl.ds(base, _C)], idx_v)
    laneoff = lax.iota(jnp.int32, 16) * _COPY

    def group(g, carry):
        off = g * 16
        bases = [idx_v[s, pl.ds(off, 16)] * EMB + laneoff for s in range(10)]
        for f in range(EMB):
            acc_r = plsc.load_gather(table_v, [bases[0] + f])
            for s in range(1, 5):
                acc_r = acc_r + plsc.load_gather(table_v, [bases[s] + f])
            acc_d = plsc.load_gather(table_v, [bases[5] + f])
            for s in range(6, 10):
                acc_d = acc_d + plsc.load_gather(table_v, [bases[s] + f])
            out_v[f, pl.ds(off, 16)] = acc_r
            out_v[EMB + f, pl.ds(off, 16)] = acc_d
        return carry

    lax.fori_loop(0, _G, group, 0)
    pltpu.sync_copy(out_v, out_hbm.at[:, pl.ds(base, _C)])


@functools.cache
def _sc_pool():
    return pl.kernel(
        _sc_pool_body,
        out_type=jax.ShapeDtypeStruct((2 * EMB, B), jnp.float32),
        mesh=plsc.VectorSubcoreMesh(
            core_axis_name="c", subcore_axis_name="s", num_cores=_NC, num_subcores=_NS
        ),
        scratch_types=[
            pltpu.VMEM((10, _C), jnp.int32),
            pltpu.VMEM((_TWORDS,), jnp.float32),
            pltpu.VMEM((2 * EMB, _C), jnp.float32),
        ],
        compiler_params=pltpu.CompilerParams(needs_layout_passes=False),
    )


_BLK = 2048


def _mlp_body(x_ref, w1t_ref, b1_ref, w2t_ref, b2_ref, w3t_ref, b3_ref, o_ref):
    x = x_ref[...]
    h = jnp.dot(w1t_ref[...], x, preferred_element_type=jnp.float32) + b1_ref[...]
    h = jnp.maximum(h, 0.0)
    h = jnp.dot(w2t_ref[...], h, preferred_element_type=jnp.float32) + b2_ref[...]
    h = jnp.maximum(h, 0.0)
    y = jnp.dot(w3t_ref[...], h, preferred_element_type=jnp.float32) + b3_ref[...]
    o_ref[...] = 1.0 / (1.0 + jnp.exp(-y))


def _mlp(xT, W1T, b1c, W2T, b2c, W3T, b3c):
    grid = B // _BLK
    full = lambda shape: pl.BlockSpec(shape, lambda i: (0, 0))
    return pl.pallas_call(
        _mlp_body,
        grid=(grid,),
        in_specs=[
            pl.BlockSpec((2 * EMB, _BLK), lambda i: (0, i)),
            full(W1T.shape), full(b1c.shape),
            full(W2T.shape), full(b2c.shape),
            full(W3T.shape), full(b3c.shape),
        ],
        out_specs=pl.BlockSpec((1, _BLK), lambda i: (0, i)),
        out_shape=jax.ShapeDtypeStruct((1, B), jnp.float32),
    )(xT, W1T, b1c, W2T, b2c, W3T, b3c)


def kernel(dire_heros, radiant_heros, embed_table, W1, b1, W2, b2, W3, b3):
    idxT = jnp.concatenate(
        [radiant_heros.astype(jnp.int32).T, dire_heros.astype(jnp.int32).T], axis=0
    )
    table_rep = jnp.tile(jnp.append(embed_table.reshape(-1), 0.0), 16)
    xT = _sc_pool()(idxT, table_rep)
    y = _mlp(xT, W1.T, b1[:, None], W2.T, b2[:, None], W3.T, b3[:, None])
    return y.T
